# baseline (speedup 1.0000x reference)
import jax
import jax.numpy as jnp
from jax.experimental import pallas as pl
from jax.experimental.pallas import tpu as pltpu


def _gcn_kernel(adj0_ref, adj1_ref, x_ref, w_ref, b_ref, out_ref, support_ref):
    i = pl.program_id(0)

    @pl.when(i == 0)
    def _():
        support_ref[...] = jnp.maximum(
            jnp.dot(x_ref[...], w_ref[...], preferred_element_type=jnp.float32), 0.0
        )

    s = support_ref[...]
    bi = adj0_ref.shape[0]
    out_ref[0:bi, :] = (
        jnp.dot(adj0_ref[...], s, preferred_element_type=jnp.float32) + b_ref[...]
    )
    out_ref[bi : 2 * bi, :] = (
        jnp.dot(adj1_ref[...], s, preferred_element_type=jnp.float32) + b_ref[...]
    )


def kernel(input, adj, gn_func, nn_func, weight, bias):
    x = input
    n, d_in = x.shape
    d_out = weight.shape[1]
    bi = 200 if n % 400 == 0 else n
    ni = n // (2 * bi) if n % 400 == 0 else 1
    b2 = bias.reshape(1, d_out).astype(jnp.float32)

    out = pl.pallas_call(
        _gcn_kernel,
        grid=(ni,),
        in_specs=[
            pl.BlockSpec((bi, n), lambda i: (2 * i, 0)),
            pl.BlockSpec((bi, n), lambda i: (2 * i + 1, 0)),
            pl.BlockSpec((n, d_in), lambda i: (0, 0)),
            pl.BlockSpec((d_in, d_out), lambda i: (0, 0)),
            pl.BlockSpec((1, d_out), lambda i: (0, 0)),
        ],
        out_specs=pl.BlockSpec((2 * bi, d_out), lambda i: (i, 0)),
        out_shape=jax.ShapeDtypeStruct((n, d_out), jnp.float32),
        scratch_shapes=[pltpu.VMEM((n, d_out), jnp.float32)],
    )(adj, adj, x, weight, b2)
    return out


# speedup vs baseline: 1.0042x; 1.0042x over previous
import jax
import jax.numpy as jnp
from jax.experimental import pallas as pl
from jax.experimental.pallas import tpu as pltpu


def _gcn_kernel(adj0_ref, adj1_ref, x_ref, w_ref, b_ref, out_ref, support_ref):
    i = pl.program_id(0)

    @pl.when(i == 0)
    def _():
        support_ref[...] = jnp.maximum(
            jnp.dot(
                x_ref[...].astype(jnp.bfloat16),
                w_ref[...].astype(jnp.bfloat16),
                preferred_element_type=jnp.float32,
            ),
            0.0,
        )

    s = support_ref[...]
    bi = adj0_ref.shape[0]
    out_ref[0:bi, :] = (
        jnp.dot(adj0_ref[...], s, preferred_element_type=jnp.float32) + b_ref[...]
    )
    out_ref[bi : 2 * bi, :] = (
        jnp.dot(adj1_ref[...], s, preferred_element_type=jnp.float32) + b_ref[...]
    )


def kernel(input, adj, gn_func, nn_func, weight, bias):
    x = input
    n, d_in = x.shape
    d_out = weight.shape[1]
    bi = 200 if n % 400 == 0 else n
    ni = n // (2 * bi) if n % 400 == 0 else 1
    b2 = bias.reshape(1, d_out).astype(jnp.float32)

    out = pl.pallas_call(
        _gcn_kernel,
        grid=(ni,),
        in_specs=[
            pl.BlockSpec((bi, n), lambda i: (2 * i, 0)),
            pl.BlockSpec((bi, n), lambda i: (2 * i + 1, 0)),
            pl.BlockSpec((n, d_in), lambda i: (0, 0)),
            pl.BlockSpec((d_in, d_out), lambda i: (0, 0)),
            pl.BlockSpec((1, d_out), lambda i: (0, 0)),
        ],
        out_specs=pl.BlockSpec((2 * bi, d_out), lambda i: (i, 0)),
        out_shape=jax.ShapeDtypeStruct((n, d_out), jnp.float32),
        scratch_shapes=[pltpu.VMEM((n, d_out), jnp.float32)],
    )(adj, adj, x, weight, b2)
    return out
